# baseline (speedup 1.0000x reference)
import functools
import jax
import jax.numpy as jnp
from jax import lax
from jax.experimental import pallas as pl
from jax.experimental.pallas import tpu as pltpu

N = 8192
HID = 512
HID2 = 64
K = 32
DELTA = 0.2
RB = 128
CB = 1024
HB = 64


def _norm_body(ea_ref, eb_ref, ia_ref, ib_ref):
    ea = ea_ref[...]
    eb = eb_ref[...]
    ia_ref[...] = lax.rsqrt(jnp.sum(ea * ea, axis=1, keepdims=True) + 1.0)
    ib_ref[...] = lax.rsqrt(jnp.sum(eb * eb, axis=1, keepdims=True) + 1.0)


def _sim_body(ea_ref, eb_ref, ia_ref, ib_ref, ay_ref, by_ref,
              sim_ref, m_ref, z_ref, m_run, z_run):
    j = pl.program_id(1)
    dot = jnp.dot(ea_ref[...], eb_ref[...].T, preferred_element_type=jnp.float32)
    match = (ay_ref[...] == by_ref[...]).astype(jnp.float32)
    sim = (dot + match) * ia_ref[...] * ib_ref[...].T
    sim_ref[...] = sim

    bm = jnp.max(sim, axis=1, keepdims=True)

    @pl.when(j == 0)
    def _():
        m_run[...] = jnp.full_like(m_run[...], -jnp.inf)
        z_run[...] = jnp.zeros_like(z_run[...])

    m_old = m_run[...]
    m_new = jnp.maximum(m_old, bm)
    z_run[...] = z_run[...] * jnp.exp(m_old - m_new) + jnp.sum(
        jnp.exp(sim - m_new), axis=1, keepdims=True)
    m_run[...] = m_new
    m_ref[...] = m_new
    z_ref[...] = z_run[...]


def _head_body(vals_ref, match_ref, m_ref, z_ref, g_ref, ea_ref,
               wlin_ref, blin_ref, wmu_ref, bmu_ref, out_ref):
    vals = vals_ref[...]
    m = m_ref[...]
    z = z_ref[...]
    v = jnp.exp(vals - m) / z
    s = 0.875 + 0.125 * match_ref[...]
    logits = s * v
    w = jnp.exp(logits - jnp.max(logits, axis=1, keepdims=True))
    b_uv = w / jnp.sum(w, axis=1, keepdims=True)
    mse = jnp.sum(b_uv[..., None] * g_ref[...], axis=1)
    h = ea_ref[...] + DELTA * (
        jnp.dot(mse, wlin_ref[...].T, preferred_element_type=jnp.float32)
        + blin_ref[...])
    out_ref[...] = (jnp.dot(h, wmu_ref[...].T, preferred_element_type=jnp.float32)
                    + bmu_ref[...])


@jax.jit
def kernel(embedding_a, embedding_b, a_y, b_y, train_batch_id,
           W_lin, b_lin, W_mu_q, b_mu_q, W_ls_q, b_ls_q,
           W_mu_p, b_mu_p, W_ls_p, b_ls_p):
    del train_batch_id, W_ls_q, b_ls_q, W_mu_p, b_mu_p, W_ls_p, b_ls_p
    a_y = a_y.astype(jnp.int32)
    b_y = b_y.astype(jnp.int32)

    inv_a, inv_b = pl.pallas_call(
        _norm_body,
        grid=(8,),
        in_specs=[pl.BlockSpec((N // 8, HID), lambda i: (i, 0)),
                  pl.BlockSpec((N // 8, HID), lambda i: (i, 0))],
        out_specs=[pl.BlockSpec((N // 8, 1), lambda i: (i, 0)),
                   pl.BlockSpec((N // 8, 1), lambda i: (i, 0))],
        out_shape=[jax.ShapeDtypeStruct((N, 1), jnp.float32),
                   jax.ShapeDtypeStruct((N, 1), jnp.float32)],
    )(embedding_a, embedding_b)

    sim, m, z = pl.pallas_call(
        _sim_body,
        grid=(N // RB, N // CB),
        in_specs=[
            pl.BlockSpec((RB, HID), lambda i, j: (i, 0)),
            pl.BlockSpec((CB, HID), lambda i, j: (j, 0)),
            pl.BlockSpec((RB, 1), lambda i, j: (i, 0)),
            pl.BlockSpec((CB, 1), lambda i, j: (j, 0)),
            pl.BlockSpec((RB, 1), lambda i, j: (i, 0)),
            pl.BlockSpec((1, CB), lambda i, j: (0, j)),
        ],
        out_specs=[
            pl.BlockSpec((RB, CB), lambda i, j: (i, j)),
            pl.BlockSpec((RB, 1), lambda i, j: (i, 0)),
            pl.BlockSpec((RB, 1), lambda i, j: (i, 0)),
        ],
        out_shape=[jax.ShapeDtypeStruct((N, N), jnp.float32),
                   jax.ShapeDtypeStruct((N, 1), jnp.float32),
                   jax.ShapeDtypeStruct((N, 1), jnp.float32)],
        scratch_shapes=[pltpu.VMEM((RB, 1), jnp.float32),
                        pltpu.VMEM((RB, 1), jnp.float32)],
        compiler_params=pltpu.CompilerParams(
            dimension_semantics=("arbitrary", "arbitrary")),
    )(embedding_a, embedding_b, inv_a, inv_b,
      a_y.reshape(N, 1), b_y.reshape(1, N))

    vals, idx = lax.top_k(sim, K)
    gathered = embedding_b[idx]
    match_k = (a_y[:, None] == b_y[idx]).astype(jnp.float32)

    z_q = pl.pallas_call(
        _head_body,
        grid=(N // HB,),
        in_specs=[
            pl.BlockSpec((HB, K), lambda i: (i, 0)),
            pl.BlockSpec((HB, K), lambda i: (i, 0)),
            pl.BlockSpec((HB, 1), lambda i: (i, 0)),
            pl.BlockSpec((HB, 1), lambda i: (i, 0)),
            pl.BlockSpec((HB, K, HID), lambda i: (i, 0, 0)),
            pl.BlockSpec((HB, HID), lambda i: (i, 0)),
            pl.BlockSpec((HID, HID), lambda i: (0, 0)),
            pl.BlockSpec((1, HID), lambda i: (0, 0)),
            pl.BlockSpec((HID2, HID), lambda i: (0, 0)),
            pl.BlockSpec((1, HID2), lambda i: (0, 0)),
        ],
        out_specs=pl.BlockSpec((HB, HID2), lambda i: (i, 0)),
        out_shape=jax.ShapeDtypeStruct((N, HID2), jnp.float32),
    )(vals, match_k, m, z, gathered, embedding_a,
      W_lin, b_lin.reshape(1, HID), W_mu_q, b_mu_q.reshape(1, HID2))
    return z_q


# speedup vs baseline: 4.5998x; 4.5998x over previous
import functools
import jax
import jax.numpy as jnp
from jax import lax
from jax.experimental import pallas as pl
from jax.experimental.pallas import tpu as pltpu
from jax.experimental.pallas import tpu_sc as plsc

N = 8192
HID = 512
HID2 = 64
K = 32
DELTA = 0.2
RB = 128
CB = 2048
G = 16
NG = N // G
NW = N // 32
NWORKERS = 32
RPW = N // NWORKERS


def _norm_body(ea_ref, eb_ref, ia_ref, ib_ref):
    ea = ea_ref[...]
    eb = eb_ref[...]
    ia_ref[...] = lax.rsqrt(jnp.sum(ea * ea, axis=1, keepdims=True) + 1.0)
    ib_ref[...] = lax.rsqrt(jnp.sum(eb * eb, axis=1, keepdims=True) + 1.0)


def _sim_body(ea_ref, eb_ref, ia_ref, ib_ref, ay_ref, by_ref,
              sim_ref, m_ref, z_ref, gm_ref, m_run, z_run):
    j = pl.program_id(1)
    dot = jnp.dot(ea_ref[...], eb_ref[...].T, preferred_element_type=jnp.float32)
    match = (ay_ref[...] == by_ref[...]).astype(jnp.float32)
    sim = (dot + match) * ia_ref[...] * ib_ref[...].T
    sim_ref[...] = sim
    gm_ref[...] = jnp.max(sim.reshape(RB, CB // G, G), axis=2)

    bm = jnp.max(sim, axis=1, keepdims=True)

    @pl.when(j == 0)
    def _():
        m_run[...] = jnp.full_like(m_run[...], -jnp.inf)
        z_run[...] = jnp.zeros_like(z_run[...])

    m_old = m_run[...]
    m_new = jnp.maximum(m_old, bm)
    z_run[...] = z_run[...] * jnp.exp(m_old - m_new) + jnp.sum(
        jnp.exp(sim - m_new), axis=1, keepdims=True)
    m_run[...] = m_new
    m_ref[...] = m_new
    z_ref[...] = z_run[...]


def _scalarize(x):
    return x if getattr(x, "ndim", 0) == 0 else jnp.max(x)


def _sc_select_body(sim_hbm, gm_hbm, mask_hbm, simbuf, gmbuf, wordbuf, sem):
    c = lax.axis_index("c")
    s = lax.axis_index("s")
    wid = s * 2 + c
    row0 = wid * RPW
    liota = lax.iota(jnp.int32, 16)
    NEG = jnp.float32(-3.0)
    BIG = jnp.int32(1 << 20)

    _gdn = lax.GatherDimensionNumbers(
        offset_dims=(), collapsed_slice_dims=(0,), start_index_map=(0,))

    def shuf(v, idx):
        return lax.gather(v, idx[:, None], dimension_numbers=_gdn,
                          slice_sizes=(1,),
                          mode=lax.GatherScatterMode.PROMISE_IN_BOUNDS)

    def bmax(v):
        for sh in (1, 2, 4, 8):
            v = jnp.maximum(v, shuf(v, liota ^ sh))
        return v

    def bmin(v):
        for sh in (1, 2, 4, 8):
            v = jnp.minimum(v, shuf(v, liota ^ sh))
        return v

    def row_body(r, carry):
        row = row0 + r
        pltpu.sync_copy(sim_hbm.at[row], simbuf)
        pltpu.sync_copy(gm_hbm.at[row], gmbuf)

        sup0 = jnp.full((16,), NEG)
        sup1 = jnp.full((16,), NEG)
        for t in range(32):
            mx = bmax(gmbuf[pl.ds(t * 16, 16)])
            if t < 16:
                sup0 = jnp.where(liota == t, mx, sup0)
            else:
                sup1 = jnp.where(liota == (t - 16), mx, sup1)

        idxv = [jnp.zeros((16,), jnp.int32), jnp.zeros((16,), jnp.int32)]
        for k in range(K):
            g = bmax(jnp.maximum(sup0, sup1))
            cand = jnp.minimum(jnp.where(sup0 == g, liota, BIG),
                               jnp.where(sup1 == g, liota + 16, BIG))
            vsel = bmin(cand)
            gmv = plsc.load_gather(gmbuf, [vsel * 16 + liota])
            lane = bmin(jnp.where(gmv == g, liota, BIG))
            grp = vsel * 16 + lane
            sv = plsc.load_gather(simbuf, [grp * 16 + liota])
            lane2 = bmin(jnp.where(sv == g, liota, BIG))
            gidx = grp * 16 + lane2
            sv2 = jnp.where(liota == lane2, NEG, sv)
            plsc.store_scatter(simbuf, [grp * 16 + liota], sv2)
            ngm = bmax(sv2)
            gmv2 = jnp.where(liota == lane, ngm, gmv)
            plsc.store_scatter(gmbuf, [vsel * 16 + liota], gmv2)
            nvm = bmax(gmv2)
            sup0 = jnp.where(liota == vsel, nvm, sup0)
            sup1 = jnp.where(liota == (vsel - 16), nvm, sup1)
            h, l = divmod(k, 16)
            idxv[h] = jnp.where(liota == l, gidx, idxv[h])

        for t in range(NW // 16):
            wordbuf[pl.ds(t * 16, 16)] = jnp.zeros((16,), jnp.int32)
        for h in range(2):
            word = lax.shift_right_logical(idxv[h], 5)
            bit = lax.shift_left(jnp.ones((16,), jnp.int32), idxv[h] & 31)
            plsc.addupdate_scatter(wordbuf, [word], bit)
        pltpu.sync_copy(wordbuf, mask_hbm.at[row])
        return carry

    lax.fori_loop(0, RPW, row_body, 0)


def _combine_body(sim_ref, sel_ref, m_ref, z_ref, ay_ref, by_ref, eb_ref,
                  ea_ref, wlin_ref, blin_ref, wmu_ref, bmu_ref,
                  out_ref, numer, denom, selx):
    j = pl.program_id(1)
    nj = pl.num_programs(1)

    @pl.when(j == 0)
    def _():
        numer[...] = jnp.zeros_like(numer[...])
        denom[...] = jnp.zeros_like(denom[...])
        words = sel_ref[...]
        wexp = jnp.broadcast_to(words[:, :, None], (RB, NW, 32)).reshape(RB, N)
        shifts = lax.broadcasted_iota(jnp.int32, (1, N), 1) & 31
        selx[...] = (lax.shift_right_logical(wexp, shifts) & 1).astype(
            jnp.float32)

    sim = sim_ref[...]
    sel = selx[:, pl.ds(j * CB, CB)]
    match = (ay_ref[...] == by_ref[...]).astype(jnp.float32)
    sfac = 0.875 + 0.125 * match
    v = jnp.exp(sim - m_ref[...]) / z_ref[...]
    w = jnp.exp(sfac * v) * sel
    numer[...] += jnp.dot(w, eb_ref[...], preferred_element_type=jnp.float32)
    denom[...] += jnp.sum(w, axis=1, keepdims=True)

    @pl.when(j == nj - 1)
    def _():
        mse = numer[...] / denom[...]
        h = ea_ref[...] + DELTA * (
            jnp.dot(mse, wlin_ref[...].T, preferred_element_type=jnp.float32)
            + blin_ref[...])
        out_ref[...] = (jnp.dot(h, wmu_ref[...].T,
                                preferred_element_type=jnp.float32)
                        + bmu_ref[...])


@jax.jit
def kernel(embedding_a, embedding_b, a_y, b_y, train_batch_id,
           W_lin, b_lin, W_mu_q, b_mu_q, W_ls_q, b_ls_q,
           W_mu_p, b_mu_p, W_ls_p, b_ls_p):
    del train_batch_id, W_ls_q, b_ls_q, W_mu_p, b_mu_p, W_ls_p, b_ls_p
    a_y = a_y.astype(jnp.int32).reshape(N, 1)
    b_y = b_y.astype(jnp.int32).reshape(1, N)

    inv_a, inv_b = pl.pallas_call(
        _norm_body,
        grid=(8,),
        in_specs=[pl.BlockSpec((N // 8, HID), lambda i: (i, 0)),
                  pl.BlockSpec((N // 8, HID), lambda i: (i, 0))],
        out_specs=[pl.BlockSpec((N // 8, 1), lambda i: (i, 0)),
                   pl.BlockSpec((N // 8, 1), lambda i: (i, 0))],
        out_shape=[jax.ShapeDtypeStruct((N, 1), jnp.float32),
                   jax.ShapeDtypeStruct((N, 1), jnp.float32)],
    )(embedding_a, embedding_b)

    sim, m, z, gm = pl.pallas_call(
        _sim_body,
        grid=(N // RB, N // CB),
        in_specs=[
            pl.BlockSpec((RB, HID), lambda i, j: (i, 0)),
            pl.BlockSpec((CB, HID), lambda i, j: (j, 0)),
            pl.BlockSpec((RB, 1), lambda i, j: (i, 0)),
            pl.BlockSpec((CB, 1), lambda i, j: (j, 0)),
            pl.BlockSpec((RB, 1), lambda i, j: (i, 0)),
            pl.BlockSpec((1, CB), lambda i, j: (0, j)),
        ],
        out_specs=[
            pl.BlockSpec((RB, CB), lambda i, j: (i, j)),
            pl.BlockSpec((RB, 1), lambda i, j: (i, 0)),
            pl.BlockSpec((RB, 1), lambda i, j: (i, 0)),
            pl.BlockSpec((RB, CB // G), lambda i, j: (i, j)),
        ],
        out_shape=[jax.ShapeDtypeStruct((N, N), jnp.float32),
                   jax.ShapeDtypeStruct((N, 1), jnp.float32),
                   jax.ShapeDtypeStruct((N, 1), jnp.float32),
                   jax.ShapeDtypeStruct((N, NG), jnp.float32)],
        scratch_shapes=[pltpu.VMEM((RB, 1), jnp.float32),
                        pltpu.VMEM((RB, 1), jnp.float32)],
        compiler_params=pltpu.CompilerParams(
            dimension_semantics=("arbitrary", "arbitrary")),
    )(embedding_a, embedding_b, inv_a, inv_b, a_y, b_y)

    selmask = pl.kernel(
        _sc_select_body,
        out_type=jax.ShapeDtypeStruct((N, NW), jnp.int32),
        mesh=plsc.VectorSubcoreMesh(core_axis_name="c", subcore_axis_name="s"),
        scratch_types=[
            pltpu.VMEM((N,), jnp.float32),
            pltpu.VMEM((NG,), jnp.float32),
            pltpu.VMEM((NW,), jnp.int32),
            pltpu.SemaphoreType.DMA,
        ],
        compiler_params=pltpu.CompilerParams(needs_layout_passes=False),
    )(sim, gm)

    z_q = pl.pallas_call(
        _combine_body,
        grid=(N // RB, N // CB),
        in_specs=[
            pl.BlockSpec((RB, CB), lambda i, j: (i, j)),
            pl.BlockSpec((RB, NW), lambda i, j: (i, 0)),
            pl.BlockSpec((RB, 1), lambda i, j: (i, 0)),
            pl.BlockSpec((RB, 1), lambda i, j: (i, 0)),
            pl.BlockSpec((RB, 1), lambda i, j: (i, 0)),
            pl.BlockSpec((1, CB), lambda i, j: (0, j)),
            pl.BlockSpec((CB, HID), lambda i, j: (j, 0)),
            pl.BlockSpec((RB, HID), lambda i, j: (i, 0)),
            pl.BlockSpec((HID, HID), lambda i, j: (0, 0)),
            pl.BlockSpec((1, HID), lambda i, j: (0, 0)),
            pl.BlockSpec((HID2, HID), lambda i, j: (0, 0)),
            pl.BlockSpec((1, HID2), lambda i, j: (0, 0)),
        ],
        out_specs=pl.BlockSpec((RB, HID2), lambda i, j: (i, 0)),
        out_shape=jax.ShapeDtypeStruct((N, HID2), jnp.float32),
        scratch_shapes=[pltpu.VMEM((RB, HID), jnp.float32),
                        pltpu.VMEM((RB, 1), jnp.float32),
                        pltpu.VMEM((RB, N), jnp.float32)],
        compiler_params=pltpu.CompilerParams(
            dimension_semantics=("arbitrary", "arbitrary")),
    )(sim, selmask, m, z, a_y, b_y, embedding_b, embedding_a,
      W_lin, b_lin.reshape(1, HID), W_mu_q, b_mu_q.reshape(1, HID2))
    return z_q


# speedup vs baseline: 7.3536x; 1.5987x over previous
import functools
import jax
import jax.numpy as jnp
from jax import lax
from jax.experimental import pallas as pl
from jax.experimental.pallas import tpu as pltpu
from jax.experimental.pallas import tpu_sc as plsc

N = 8192
HID = 512
HID2 = 64
K = 32
DELTA = 0.2
RB = 128
CB = 2048
G = 16
NG = N // G
NW = N // 32
NWORKERS = 32
RPW = N // NWORKERS


def _norm_body(ea_ref, eb_ref, ia_ref, ib_ref):
    ea = ea_ref[...]
    eb = eb_ref[...]
    ia_ref[...] = lax.rsqrt(jnp.sum(ea * ea, axis=1, keepdims=True) + 1.0)
    ib_ref[...] = lax.rsqrt(jnp.sum(eb * eb, axis=1, keepdims=True) + 1.0)


def _win16max(x):
    for sh in (1, 2, 4, 8):
        x = jnp.maximum(x, jnp.roll(x, -sh, axis=1))
    return x


def _sim_body(ea_ref, eb_ref, ia_ref, ib_ref, ay_ref, by_ref, s1_ref, s2_ref,
              sim_ref, m_ref, z_ref, gm_ref, sm_ref, m_run, z_run):
    j = pl.program_id(1)
    dot = jnp.dot(ea_ref[...], eb_ref[...].T, preferred_element_type=jnp.float32)
    match = (ay_ref[...] == by_ref[...]).astype(jnp.float32)
    sim = (dot + match) * ia_ref[...] * ib_ref[...].T
    sim_ref[...] = sim
    gm = jnp.dot(_win16max(sim), s1_ref[...], preferred_element_type=jnp.float32,
                 precision=lax.Precision.HIGHEST)
    gm_ref[:, pl.ds(j * (CB // G), CB // G)] = gm

    @pl.when(j == pl.num_programs(1) - 1)
    def _():
        sm_ref[...] = jnp.dot(_win16max(gm_ref[...]), s2_ref[...],
                              preferred_element_type=jnp.float32,
                              precision=lax.Precision.HIGHEST)

    bm = jnp.max(gm, axis=1, keepdims=True)

    @pl.when(j == 0)
    def _():
        m_run[...] = jnp.full_like(m_run[...], -jnp.inf)
        z_run[...] = jnp.zeros_like(z_run[...])

    m_old = m_run[...]
    m_new = jnp.maximum(m_old, bm)
    z_run[...] = z_run[...] * jnp.exp(m_old - m_new) + jnp.sum(
        jnp.exp(sim - m_new), axis=1, keepdims=True)
    m_run[...] = m_new
    m_ref[...] = m_new
    z_ref[...] = z_run[...]


def _scalarize(x):
    return x if getattr(x, "ndim", 0) == 0 else jnp.max(x)


def _sc_select_body(sim_hbm, gm_hbm, sm_hbm, mask_hbm,
                    simbuf0, simbuf1, gmbuf0, gmbuf1, smbuf0, smbuf1,
                    wordbuf, ssem0, ssem1, gsem0, gsem1, msem0, msem1):
    simbuf = (simbuf0, simbuf1)
    gmbuf = (gmbuf0, gmbuf1)
    smbuf = (smbuf0, smbuf1)
    ssem = (ssem0, ssem1)
    gsem = (gsem0, gsem1)
    msem = (msem0, msem1)
    c = lax.axis_index("c")
    s = lax.axis_index("s")
    wid = s * 2 + c
    row0 = wid * RPW
    liota = lax.iota(jnp.int32, 16)
    NEG = jnp.float32(-3.0)

    _gdn = lax.GatherDimensionNumbers(
        offset_dims=(), collapsed_slice_dims=(0,), start_index_map=(0,))

    def shuf(v, idx):
        return lax.gather(v, idx[:, None], dimension_numbers=_gdn,
                          slice_sizes=(1,),
                          mode=lax.GatherScatterMode.PROMISE_IN_BOUNDS)

    def bmax(v):
        for sh in (1, 2, 4, 8):
            v = jnp.maximum(v, shuf(v, liota ^ sh))
        return v

    def start(row, slot):
        pltpu.make_async_copy(sim_hbm.at[row], simbuf[slot], ssem[slot]).start()
        pltpu.make_async_copy(gm_hbm.at[row], gmbuf[slot], gsem[slot]).start()
        pltpu.make_async_copy(sm_hbm.at[row], smbuf[slot], msem[slot]).start()

    def wait(row, slot):
        pltpu.make_async_copy(sim_hbm.at[row], simbuf[slot], ssem[slot]).wait()
        pltpu.make_async_copy(gm_hbm.at[row], gmbuf[slot], gsem[slot]).wait()
        pltpu.make_async_copy(sm_hbm.at[row], smbuf[slot], msem[slot]).wait()

    def process(row, sbuf, gbuf, mbuf):
        sup0 = mbuf[pl.ds(0, 16)]
        sup1 = mbuf[pl.ds(16, 16)]
        idxv = [jnp.zeros((16,), jnp.int32), jnp.zeros((16,), jnp.int32)]
        for k in range(K):
            g = bmax(jnp.maximum(sup0, sup1))
            f0 = plsc.all_reduce_ffs(sup0 == g)
            f1 = plsc.all_reduce_ffs(sup1 == g)
            vsel = jnp.where(f0 < 16, f0, f1 + 16)
            gmv = plsc.load_gather(gbuf, [vsel * 16 + liota])
            lane = plsc.all_reduce_ffs(gmv == g)
            grp = vsel * 16 + lane
            sv = plsc.load_gather(sbuf, [grp * 16 + liota])
            lane2 = plsc.all_reduce_ffs(sv == g)
            gidx = grp * 16 + lane2
            sv2 = jnp.where(liota == lane2, NEG, sv)
            plsc.store_scatter(sbuf, [grp * 16 + liota], sv2)
            ngm = bmax(sv2)
            gmv2 = jnp.where(liota == lane, ngm, gmv)
            plsc.store_scatter(gbuf, [vsel * 16 + liota], gmv2)
            nvm = bmax(gmv2)
            sup0 = jnp.where(liota == vsel, nvm, sup0)
            sup1 = jnp.where(liota == (vsel - 16), nvm, sup1)
            h, l = divmod(k, 16)
            idxv[h] = jnp.where(liota == l, gidx, idxv[h])

        for t in range(NW // 16):
            wordbuf[pl.ds(t * 16, 16)] = jnp.zeros((16,), jnp.int32)
        for h in range(2):
            word = lax.shift_right_logical(idxv[h], 5)
            bit = lax.shift_left(jnp.ones((16,), jnp.int32), idxv[h] & 31)
            plsc.addupdate_scatter(wordbuf, [word], bit)
        pltpu.sync_copy(wordbuf, mask_hbm.at[row])

    start(row0, 0)

    def pair_body(p, carry):
        ra = row0 + 2 * p
        rb = ra + 1
        wait(ra, 0)
        start(rb, 1)
        process(ra, simbuf[0], gmbuf[0], smbuf[0])
        wait(rb, 1)

        @pl.when(rb + 1 < row0 + RPW)
        def _():
            start(rb + 1, 0)

        process(rb, simbuf[1], gmbuf[1], smbuf[1])
        return carry

    lax.fori_loop(0, RPW // 2, pair_body, 0)


def _combine_body(sim_ref, sel_ref, m_ref, z_ref, ay_ref, by_ref, eb_ref,
                  ea_ref, wlin_ref, blin_ref, wmu_ref, bmu_ref,
                  out_ref, numer, denom, selx):
    j = pl.program_id(1)
    nj = pl.num_programs(1)

    @pl.when(j == 0)
    def _():
        numer[...] = jnp.zeros_like(numer[...])
        denom[...] = jnp.zeros_like(denom[...])
        words = sel_ref[...]
        wexp = jnp.broadcast_to(words[:, :, None], (RB, NW, 32)).reshape(RB, N)
        shifts = lax.broadcasted_iota(jnp.int32, (1, N), 1) & 31
        selx[...] = (lax.shift_right_logical(wexp, shifts) & 1).astype(
            jnp.float32)

    sim = sim_ref[...]
    sel = selx[:, pl.ds(j * CB, CB)]
    match = (ay_ref[...] == by_ref[...]).astype(jnp.float32)
    sfac = 0.875 + 0.125 * match
    v = jnp.exp(sim - m_ref[...]) / z_ref[...]
    w = jnp.exp(sfac * v) * sel
    numer[...] += jnp.dot(w, eb_ref[...], preferred_element_type=jnp.float32)
    denom[...] += jnp.sum(w, axis=1, keepdims=True)

    @pl.when(j == nj - 1)
    def _():
        mse = numer[...] / denom[...]
        h = ea_ref[...] + DELTA * (
            jnp.dot(mse, wlin_ref[...].T, preferred_element_type=jnp.float32)
            + blin_ref[...])
        out_ref[...] = (jnp.dot(h, wmu_ref[...].T,
                                preferred_element_type=jnp.float32)
                        + bmu_ref[...])


@jax.jit
def kernel(embedding_a, embedding_b, a_y, b_y, train_batch_id,
           W_lin, b_lin, W_mu_q, b_mu_q, W_ls_q, b_ls_q,
           W_mu_p, b_mu_p, W_ls_p, b_ls_p):
    del train_batch_id, W_ls_q, b_ls_q, W_mu_p, b_mu_p, W_ls_p, b_ls_p
    a_y = a_y.astype(jnp.int32).reshape(N, 1)
    b_y = b_y.astype(jnp.int32).reshape(1, N)

    inv_a, inv_b = pl.pallas_call(
        _norm_body,
        grid=(8,),
        in_specs=[pl.BlockSpec((N // 8, HID), lambda i: (i, 0)),
                  pl.BlockSpec((N // 8, HID), lambda i: (i, 0))],
        out_specs=[pl.BlockSpec((N // 8, 1), lambda i: (i, 0)),
                   pl.BlockSpec((N // 8, 1), lambda i: (i, 0))],
        out_shape=[jax.ShapeDtypeStruct((N, 1), jnp.float32),
                   jax.ShapeDtypeStruct((N, 1), jnp.float32)],
    )(embedding_a, embedding_b)

    gi = lax.broadcasted_iota(jnp.int32, (CB, CB // G), 0)
    gj = lax.broadcasted_iota(jnp.int32, (CB, CB // G), 1)
    s1 = (gi == G * gj).astype(jnp.float32)
    ti = lax.broadcasted_iota(jnp.int32, (NG, NG // G), 0)
    tj = lax.broadcasted_iota(jnp.int32, (NG, NG // G), 1)
    s2 = (ti == G * tj).astype(jnp.float32)

    sim, m, z, gm, sm = pl.pallas_call(
        _sim_body,
        grid=(N // RB, N // CB),
        in_specs=[
            pl.BlockSpec((RB, HID), lambda i, j: (i, 0)),
            pl.BlockSpec((CB, HID), lambda i, j: (j, 0)),
            pl.BlockSpec((RB, 1), lambda i, j: (i, 0)),
            pl.BlockSpec((CB, 1), lambda i, j: (j, 0)),
            pl.BlockSpec((RB, 1), lambda i, j: (i, 0)),
            pl.BlockSpec((1, CB), lambda i, j: (0, j)),
            pl.BlockSpec((CB, CB // G), lambda i, j: (0, 0)),
            pl.BlockSpec((NG, NG // G), lambda i, j: (0, 0)),
        ],
        out_specs=[
            pl.BlockSpec((RB, CB), lambda i, j: (i, j)),
            pl.BlockSpec((RB, 1), lambda i, j: (i, 0)),
            pl.BlockSpec((RB, 1), lambda i, j: (i, 0)),
            pl.BlockSpec((RB, NG), lambda i, j: (i, 0)),
            pl.BlockSpec((RB, NG // G), lambda i, j: (i, 0)),
        ],
        out_shape=[jax.ShapeDtypeStruct((N, N), jnp.float32),
                   jax.ShapeDtypeStruct((N, 1), jnp.float32),
                   jax.ShapeDtypeStruct((N, 1), jnp.float32),
                   jax.ShapeDtypeStruct((N, NG), jnp.float32),
                   jax.ShapeDtypeStruct((N, NG // G), jnp.float32)],
        scratch_shapes=[pltpu.VMEM((RB, 1), jnp.float32),
                        pltpu.VMEM((RB, 1), jnp.float32)],
        compiler_params=pltpu.CompilerParams(
            dimension_semantics=("arbitrary", "arbitrary")),
    )(embedding_a, embedding_b, inv_a, inv_b, a_y, b_y, s1, s2)

    selmask = pl.kernel(
        _sc_select_body,
        out_type=jax.ShapeDtypeStruct((N, NW), jnp.int32),
        mesh=plsc.VectorSubcoreMesh(core_axis_name="c", subcore_axis_name="s"),
        scratch_types=[
            pltpu.VMEM((N,), jnp.float32),
            pltpu.VMEM((N,), jnp.float32),
            pltpu.VMEM((NG,), jnp.float32),
            pltpu.VMEM((NG,), jnp.float32),
            pltpu.VMEM((NG // G,), jnp.float32),
            pltpu.VMEM((NG // G,), jnp.float32),
            pltpu.VMEM((NW,), jnp.int32),
            pltpu.SemaphoreType.DMA,
            pltpu.SemaphoreType.DMA,
            pltpu.SemaphoreType.DMA,
            pltpu.SemaphoreType.DMA,
            pltpu.SemaphoreType.DMA,
            pltpu.SemaphoreType.DMA,
        ],
        compiler_params=pltpu.CompilerParams(needs_layout_passes=False),
    )(sim, gm, sm)

    z_q = pl.pallas_call(
        _combine_body,
        grid=(N // RB, N // CB),
        in_specs=[
            pl.BlockSpec((RB, CB), lambda i, j: (i, j)),
            pl.BlockSpec((RB, NW), lambda i, j: (i, 0)),
            pl.BlockSpec((RB, 1), lambda i, j: (i, 0)),
            pl.BlockSpec((RB, 1), lambda i, j: (i, 0)),
            pl.BlockSpec((RB, 1), lambda i, j: (i, 0)),
            pl.BlockSpec((1, CB), lambda i, j: (0, j)),
            pl.BlockSpec((CB, HID), lambda i, j: (j, 0)),
            pl.BlockSpec((RB, HID), lambda i, j: (i, 0)),
            pl.BlockSpec((HID, HID), lambda i, j: (0, 0)),
            pl.BlockSpec((1, HID), lambda i, j: (0, 0)),
            pl.BlockSpec((HID2, HID), lambda i, j: (0, 0)),
            pl.BlockSpec((1, HID2), lambda i, j: (0, 0)),
        ],
        out_specs=pl.BlockSpec((RB, HID2), lambda i, j: (i, 0)),
        out_shape=jax.ShapeDtypeStruct((N, HID2), jnp.float32),
        scratch_shapes=[pltpu.VMEM((RB, HID), jnp.float32),
                        pltpu.VMEM((RB, 1), jnp.float32),
                        pltpu.VMEM((RB, N), jnp.float32)],
        compiler_params=pltpu.CompilerParams(
            dimension_semantics=("arbitrary", "arbitrary")),
    )(sim, selmask, m, z, a_y, b_y, embedding_b, embedding_a,
      W_lin, b_lin.reshape(1, HID), W_mu_q, b_mu_q.reshape(1, HID2))
    return z_q
